# baseline (speedup 1.0000x reference)
import functools

import jax
import jax.numpy as jnp
from jax import lax
from jax.experimental import pallas as pl
from jax.experimental.pallas import tpu as pltpu
from jax.experimental.pallas import tpu_sc as plsc

N = 10000
E = 320000
D = 128

NC = 2
NS = 16
L = 16
EPW = E // (NC * NS)
K = 40
NCHUNK = EPW // K
NPAD = 10112
RPT = NPAD // NS
NB = 5
GL = 3
GROUPS = ((0, 0), (16, 0), (24, 8))
assert NCHUNK % NB == 0 and RPT % 8 == 0

_DIMNUMS = lax.GatherDimensionNumbers(
    offset_dims=(), collapsed_slice_dims=(0,), start_index_map=(0,))


def _bcast_lane(v16, ii):
    return lax.gather(v16, jnp.full((L, 1), ii, jnp.int32), _DIMNUMS, (1,),
                      mode=lax.GatherScatterMode.PROMISE_IN_BOUNDS)


@functools.partial(
    pl.kernel,
    mesh=plsc.VectorSubcoreMesh(core_axis_name="c", subcore_axis_name="s"),
    compiler_params=pltpu.CompilerParams(use_tc_tiling_on_sc=False),
    out_type=jax.ShapeDtypeStruct((NC, NPAD, D), jnp.float32),
    scratch_types=[
        pltpu.VMEM_SHARED((NPAD, D), jnp.float32),
        pltpu.VMEM((NCHUNK, K), jnp.int32),
        pltpu.VMEM((NCHUNK, K), jnp.int32),
        pltpu.VMEM((NB, 1, K), jnp.float32),
        pltpu.VMEM((NB, K, D), jnp.float32),
    ] + [pltpu.SemaphoreType.DMA] * (3 * NB),
)
def _spmm_sc(x_hbm, rows4_hbm, cols4_hbm, vals5_hbm, zrow_hbm, out_hbm,
             accum, rowsv, colsv, valsb, gbuf, *sems):
    c = lax.axis_index("c")
    s = lax.axis_index("s")
    semg = sems[:NB]
    sems_ = sems[NB:2 * NB]
    semv = sems[2 * NB:]

    r0 = s * RPT
    pltpu.sync_copy(zrow_hbm, accum.at[pl.ds(r0, RPT)])

    pltpu.sync_copy(rows4_hbm.at[c, s], rowsv)
    pltpu.sync_copy(cols4_hbm.at[c, s], colsv)
    plsc.subcore_barrier()

    def scale(b, j):
        for off, lane0 in GROUPS:
            v16 = valsb[b, 0, pl.ds(off, L)]
            for ii in range(lane0, L):
                i = off + ii
                vb = _bcast_lane(v16, ii)
                for d in range(D // L):
                    gv = gbuf[b, i, pl.ds(d * L, L)]
                    gbuf[b, i, pl.ds(d * L, L)] = gv * vb

    for p in range(GL):
        pltpu.async_copy(x_hbm.at[colsv.at[p]], gbuf.at[p], semg[p])
        pltpu.async_copy(vals5_hbm.at[c, s, p], valsb.at[p], semv[p])

    def round_body(jj, _):
        for b in range(NB):
            j = jj * NB + b
            bn = (b + GL) % NB
            pltpu.make_async_copy(x_hbm.at[colsv.at[j]], gbuf.at[b],
                                  semg[b]).wait()
            pltpu.make_async_copy(vals5_hbm.at[c, s, j], valsb.at[b],
                                  semv[b]).wait()
            scale(b, j)
            pltpu.async_copy(gbuf.at[b], accum.at[rowsv.at[j]], sems_[b],
                             add=True)

            @pl.when(j >= NB - GL)
            def _drain_prev(j=j, bn=bn):
                pltpu.make_async_copy(gbuf.at[bn],
                                      accum.at[rowsv.at[j - (NB - GL)]],
                                      sems_[bn]).wait()

            @pl.when(j + GL < NCHUNK)
            def _start_next(j=j, bn=bn):
                pltpu.async_copy(x_hbm.at[colsv.at[j + GL]], gbuf.at[bn],
                                 semg[bn])
                pltpu.async_copy(vals5_hbm.at[c, s, j + GL], valsb.at[bn],
                                 semv[bn])
        return 0

    lax.fori_loop(0, NCHUNK // NB, round_body, 0)
    for j in range(NCHUNK - (NB - GL), NCHUNK):
        b = j % NB
        pltpu.make_async_copy(gbuf.at[b], accum.at[rowsv.at[j]],
                              sems_[b]).wait()
    plsc.subcore_barrier()

    pltpu.sync_copy(accum.at[pl.ds(r0, RPT)], out_hbm.at[c, pl.ds(r0, RPT)])


RBLK = 1000


def _combine_body(p_ref, b_ref, o_ref):
    o_ref[...] = p_ref[0] + p_ref[1] + b_ref[...]


_combine = pl.pallas_call(
    _combine_body,
    grid=(N // RBLK,),
    in_specs=[pl.BlockSpec((NC, RBLK, D), lambda i: (0, i, 0)),
              pl.BlockSpec((1, D), lambda i: (0, 0))],
    out_specs=pl.BlockSpec((RBLK, D), lambda i: (i, 0)),
    out_shape=jax.ShapeDtypeStruct((N, D), jnp.float32),
)


def kernel(x, L_indices, L_values, bias):
    rows4 = L_indices[0].astype(jnp.int32).reshape(NC, NS, NCHUNK, K)
    cols4 = L_indices[1].astype(jnp.int32).reshape(NC, NS, NCHUNK, K)
    vals5 = L_values.reshape(NC, NS, NCHUNK, 1, K)
    zrow = jnp.zeros((RPT, D), jnp.float32)
    partials = _spmm_sc(x, rows4, cols4, vals5, zrow)
    return _combine(partials, bias.reshape(1, D))


# speedup vs baseline: 1.0594x; 1.0594x over previous
import functools

import jax
import jax.numpy as jnp
from jax import lax
from jax.experimental import pallas as pl
from jax.experimental.pallas import tpu as pltpu
from jax.experimental.pallas import tpu_sc as plsc

N = 10000
E = 320000
D = 128

NC = 2
NS = 16
L = 16
EPW = E // (NC * NS)
K = 40
NCHUNK = EPW // K
NPAD = 10112
RPT = NPAD // NS
NB = 5
GL = 4
GROUPS = ((0, 0), (16, 0), (24, 8))
assert NCHUNK % NB == 0 and RPT % 8 == 0

_DIMNUMS = lax.GatherDimensionNumbers(
    offset_dims=(), collapsed_slice_dims=(0,), start_index_map=(0,))


def _bcast_lane(v16, ii):
    return lax.gather(v16, jnp.full((L, 1), ii, jnp.int32), _DIMNUMS, (1,),
                      mode=lax.GatherScatterMode.PROMISE_IN_BOUNDS)


@functools.partial(
    pl.kernel,
    mesh=plsc.VectorSubcoreMesh(core_axis_name="c", subcore_axis_name="s"),
    compiler_params=pltpu.CompilerParams(use_tc_tiling_on_sc=False),
    out_type=jax.ShapeDtypeStruct((NC, NPAD, D), jnp.float32),
    scratch_types=[
        pltpu.VMEM_SHARED((NPAD, D), jnp.float32),
        pltpu.VMEM((NCHUNK, K), jnp.int32),
        pltpu.VMEM((NCHUNK, K), jnp.int32),
        pltpu.VMEM((NB, 1, K), jnp.float32),
        pltpu.VMEM((NB, K, D), jnp.float32),
    ] + [pltpu.SemaphoreType.DMA] * (3 * NB),
)
def _spmm_sc(x_hbm, rows4_hbm, cols4_hbm, vals5_hbm, zrow_hbm, out_hbm,
             accum, rowsv, colsv, valsb, gbuf, *sems):
    c = lax.axis_index("c")
    s = lax.axis_index("s")
    semg = sems[:NB]
    sems_ = sems[NB:2 * NB]
    semv = sems[2 * NB:]

    r0 = s * RPT
    pltpu.sync_copy(zrow_hbm, accum.at[pl.ds(r0, RPT)])

    pltpu.sync_copy(rows4_hbm.at[c, s], rowsv)
    pltpu.sync_copy(cols4_hbm.at[c, s], colsv)
    plsc.subcore_barrier()

    def scale(b, j):
        for off, lane0 in GROUPS:
            v16 = valsb[b, 0, pl.ds(off, L)]
            for ii in range(lane0, L):
                i = off + ii
                vb = _bcast_lane(v16, ii)
                for d in range(D // L):
                    gv = gbuf[b, i, pl.ds(d * L, L)]
                    gbuf[b, i, pl.ds(d * L, L)] = gv * vb

    for p in range(GL):
        pltpu.async_copy(x_hbm.at[colsv.at[p]], gbuf.at[p], semg[p])
        pltpu.async_copy(vals5_hbm.at[c, s, p], valsb.at[p], semv[p])

    def round_body(jj, _):
        for b in range(NB):
            j = jj * NB + b
            bn = (b + GL) % NB
            pltpu.make_async_copy(x_hbm.at[colsv.at[j]], gbuf.at[b],
                                  semg[b]).wait()
            pltpu.make_async_copy(vals5_hbm.at[c, s, j], valsb.at[b],
                                  semv[b]).wait()
            scale(b, j)
            pltpu.async_copy(gbuf.at[b], accum.at[rowsv.at[j]], sems_[b],
                             add=True)

            @pl.when(j >= NB - GL)
            def _drain_prev(j=j, bn=bn):
                pltpu.make_async_copy(gbuf.at[bn],
                                      accum.at[rowsv.at[j - (NB - GL)]],
                                      sems_[bn]).wait()

            @pl.when(j + GL < NCHUNK)
            def _start_next(j=j, bn=bn):
                pltpu.async_copy(x_hbm.at[colsv.at[j + GL]], gbuf.at[bn],
                                 semg[bn])
                pltpu.async_copy(vals5_hbm.at[c, s, j + GL], valsb.at[bn],
                                 semv[bn])
        return 0

    lax.fori_loop(0, NCHUNK // NB, round_body, 0)
    for j in range(NCHUNK - (NB - GL), NCHUNK):
        b = j % NB
        pltpu.make_async_copy(gbuf.at[b], accum.at[rowsv.at[j]],
                              sems_[b]).wait()
    plsc.subcore_barrier()

    pltpu.sync_copy(accum.at[pl.ds(r0, RPT)], out_hbm.at[c, pl.ds(r0, RPT)])


RBLK = 1000


def _combine_body(p_ref, b_ref, o_ref):
    o_ref[...] = p_ref[0] + p_ref[1] + b_ref[...]


_combine = pl.pallas_call(
    _combine_body,
    grid=(N // RBLK,),
    in_specs=[pl.BlockSpec((NC, RBLK, D), lambda i: (0, i, 0)),
              pl.BlockSpec((1, D), lambda i: (0, 0))],
    out_specs=pl.BlockSpec((RBLK, D), lambda i: (i, 0)),
    out_shape=jax.ShapeDtypeStruct((N, D), jnp.float32),
)


def kernel(x, L_indices, L_values, bias):
    rows4 = L_indices[0].astype(jnp.int32).reshape(NC, NS, NCHUNK, K)
    cols4 = L_indices[1].astype(jnp.int32).reshape(NC, NS, NCHUNK, K)
    vals5 = L_values.reshape(NC, NS, NCHUNK, 1, K)
    zrow = jnp.zeros((RPT, D), jnp.float32)
    partials = _spmm_sc(x, rows4, cols4, vals5, zrow)
    return _combine(partials, bias.reshape(1, D))
